# baseline (speedup 1.0000x reference)
import jax
import jax.numpy as jnp
from jax import lax
from jax.experimental import pallas as pl
from jax.experimental.pallas import tpu as pltpu

BATCH = 4096
IN_DIM = 128
INTER_DIM = 128
VOCAB = 1000
G = 8
CH = 8
BP = BATCH + CH
RM = 512


def _grouped_matvec_body(starts_ref, dw_ref, c_ref, out_ref):
    i = pl.program_id(0)
    for j in range(G):
        v = i * G + j
        s = starts_ref[v]
        e = starts_ref[v + 1]
        w = dw_ref[j]

        def step(k):
            rows = c_ref[pl.ds(k, CH), :]
            out_ref[pl.ds(k, CH), :] = jnp.tanh(
                jax.lax.dot(rows, w, preferred_element_type=jnp.float32))
            return k + CH

        lax.while_loop(lambda k: k < e, step, s)


def _logits_body(inter_ref, lw_ref, b_ref, out_ref):
    logits = jax.lax.dot_general(
        inter_ref[...], lw_ref[...], (((1,), (1,)), ((), ())),
        preferred_element_type=jnp.float32)
    out_ref[...] = jax.nn.sigmoid(logits + b_ref[...])


@jax.jit
def kernel(vocab_ids, compressed, decoder_weights, linear_w, linear_b):
    counts = jnp.zeros((VOCAB,), jnp.int32).at[vocab_ids].add(1)
    starts = jnp.concatenate(
        [jnp.zeros((1,), jnp.int32), jnp.cumsum(counts, dtype=jnp.int32)])
    perm = jnp.argsort(vocab_ids)
    inv_perm = jnp.zeros((BATCH,), jnp.int32).at[perm].set(
        jnp.arange(BATCH, dtype=jnp.int32))
    c_sorted = jnp.zeros((BP, IN_DIM), jnp.float32).at[:BATCH].set(
        compressed[perm])

    inter_sorted = pl.pallas_call(
        _grouped_matvec_body,
        grid_spec=pltpu.PrefetchScalarGridSpec(
            num_scalar_prefetch=1,
            grid=(VOCAB // G,),
            in_specs=[
                pl.BlockSpec((G, IN_DIM, INTER_DIM), lambda i, st: (i, 0, 0)),
                pl.BlockSpec((BP, IN_DIM), lambda i, st: (0, 0)),
            ],
            out_specs=pl.BlockSpec((BP, INTER_DIM), lambda i, st: (0, 0)),
        ),
        out_shape=jax.ShapeDtypeStruct((BP, INTER_DIM), jnp.float32),
    )(starts, decoder_weights, c_sorted)

    inter = inter_sorted[inv_perm]

    out = pl.pallas_call(
        _logits_body,
        grid=(BATCH // RM,),
        in_specs=[
            pl.BlockSpec((RM, INTER_DIM), lambda i: (i, 0)),
            pl.BlockSpec((VOCAB, INTER_DIM), lambda i: (0, 0)),
            pl.BlockSpec((1, VOCAB), lambda i: (0, 0)),
        ],
        out_specs=pl.BlockSpec((RM, VOCAB), lambda i: (i, 0)),
        out_shape=jax.ShapeDtypeStruct((BATCH, VOCAB), jnp.float32),
    )(inter, linear_w, linear_b.reshape(1, VOCAB))
    return out
